# Initial kernel scaffold; baseline (speedup 1.0000x reference)
import jax
import jax.numpy as jnp
from jax.experimental import pallas as pl


def kernel(x, labels, W, b):
    raise NotImplementedError("write your pallas kernel here")


# speedup vs baseline: 1.3706x; 1.3706x over previous
import functools

import jax
import jax.numpy as jnp
import numpy as np
from jax.experimental import pallas as pl
from jax.experimental.pallas import tpu as pltpu

B, C, T = 64, 768, 1024
NUM_CLASSES = 4
DROP_NUM = int(C * 0.33)
ONE_BITS = np.float32(1.0).view(np.int32).item()


def _pass1_body(labels_ref, x_ref, W_ref, g_ref, pool_ref):
    b = pl.program_id(0)
    lab = labels_ref[b]
    X = x_ref[0]
    cls = jax.lax.broadcasted_iota(jnp.int32, (NUM_CLASSES, 1), 0)
    w = jnp.sum(jnp.where(cls == lab, W_ref[...], 0.0), axis=0)
    s = X * w[:, None]
    smax = jnp.max(s, axis=0)
    smin = jnp.min(s, axis=0)
    inv = 1.0 / (smax - smin)
    g_ref[0, 0, :] = w * jnp.sum(X * inv[None, :], axis=1)
    pool_ref[0, 0, :] = jnp.sum(X, axis=1)


def _pass2_body(g_ref, pool_ref, W_ref, bias_ref, r_ref, y_ref, mask_ref):
    g = g_ref[...]
    r = r_ref[...]
    pooled = pool_ref[...] * (1.0 / T)
    y_ref[...] = jnp.dot(pooled, W_ref[...].T,
                         preferred_element_type=jnp.float32) + bias_ref[0]
    gmin = jnp.min(g, axis=1, keepdims=True)
    gmax = jnp.max(g, axis=1, keepdims=True)
    sn = (g - gmin) / (gmax - gmin)
    key = jnp.exp(jnp.log(r) / sn)
    lo = jnp.zeros((B, 1), jnp.int32)
    hi = jnp.full((B, 1), ONE_BITS, jnp.int32)
    for _ in range(31):
        mid = (lo + hi) // 2
        midf = jax.lax.bitcast_convert_type(mid, jnp.float32)
        cnt = jnp.sum(jnp.where(key > midf, 1, 0), axis=1, keepdims=True)
        take_hi = cnt <= DROP_NUM
        hi = jnp.where(take_hi, mid, hi)
        lo = jnp.where(take_hi, lo, mid + 1)
    thr = jax.lax.bitcast_convert_type(hi, jnp.float32)
    mask_ref[...] = jnp.where(key > thr, 0.0, 1.0)


@functools.lru_cache(maxsize=1)
def _wrs_uniform():
    with jax.ensure_compile_time_eval():
        return np.asarray(
            jax.random.uniform(jax.random.key(42), (B, C), dtype=jnp.float32))


def kernel(x, labels, W, b):
    labels = labels.astype(jnp.int32)
    g3, pool3 = pl.pallas_call(
        _pass1_body,
        grid=(B,),
        in_specs=[
            pl.BlockSpec(memory_space=pltpu.SMEM),
            pl.BlockSpec((1, C, T), lambda i: (i, 0, 0)),
            pl.BlockSpec((NUM_CLASSES, C), lambda i: (0, 0)),
        ],
        out_specs=[
            pl.BlockSpec((1, 1, C), lambda i: (i, 0, 0)),
            pl.BlockSpec((1, 1, C), lambda i: (i, 0, 0)),
        ],
        out_shape=[
            jax.ShapeDtypeStruct((B, 1, C), jnp.float32),
            jax.ShapeDtypeStruct((B, 1, C), jnp.float32),
        ],
    )(labels, x, W)
    g = g3.reshape(B, C)
    pool = pool3.reshape(B, C)
    r = jnp.asarray(_wrs_uniform())
    y, mask = pl.pallas_call(
        _pass2_body,
        in_specs=[
            pl.BlockSpec((B, C), lambda: (0, 0)),
            pl.BlockSpec((B, C), lambda: (0, 0)),
            pl.BlockSpec((NUM_CLASSES, C), lambda: (0, 0)),
            pl.BlockSpec((1, NUM_CLASSES), lambda: (0, 0)),
            pl.BlockSpec((B, C), lambda: (0, 0)),
        ],
        out_specs=[
            pl.BlockSpec((B, NUM_CLASSES), lambda: (0, 0)),
            pl.BlockSpec((B, C), lambda: (0, 0)),
        ],
        out_shape=[
            jax.ShapeDtypeStruct((B, NUM_CLASSES), jnp.float32),
            jax.ShapeDtypeStruct((B, C), jnp.float32),
        ],
    )(g, pool, W, b.reshape(1, NUM_CLASSES), r)
    return (y, mask[:, :, None])
